# baseline (speedup 1.0000x reference)
import functools

import jax
import jax.numpy as jnp
from jax import lax
from jax.experimental import pallas as pl
from jax.experimental.pallas import tpu as pltpu
from jax.experimental.pallas import tpu_sc as plsc

_B = 262144
_C = 6
_RES = (8, 32, 128, 512)
_NC = 2
_NS = 16
_NW = _NC * _NS
_LANES = 16
_CHUNK = 64
_NSUB = _CHUNK // _LANES
_RPW = _B // _NW
_NCHUNK = _RPW // _CHUNK
_CP = 8
_R2 = 6 * _RES[2] * _RES[2]
_R3 = 6 * _RES[3] * _RES[3]
_T3CH = 1024
_T2CH = 1024
_N3 = (_RES[3] * _RES[3]) // (_NS * _T3CH)
_NCH3 = 6 * _N3


def _dir_math(x, y, z):
    ax, ay, az = jnp.abs(x), jnp.abs(y), jnp.abs(z)
    ma = jnp.maximum(jnp.maximum(ax, ay), az)
    is_x = (ax >= ay) & (ax >= az)
    is_y = (~is_x) & (ay >= az)
    face = jnp.where(
        is_x, jnp.where(x >= 0, 0, 1),
        jnp.where(is_y, jnp.where(y >= 0, 2, 3), jnp.where(z >= 0, 4, 5)),
    ).astype(jnp.int32)
    sc = jnp.where(is_x, jnp.where(x >= 0, -z, z),
                   jnp.where(is_y, x, jnp.where(z >= 0, x, -x)))
    tc = jnp.where(is_y, jnp.where(y >= 0, z, -z), -y)
    safe = jnp.where(ma > 0, ma, jnp.float32(1.0))
    u = 0.5 * (sc / safe + 1.0)
    v = 0.5 * (tc / safe + 1.0)
    return face, u, v, ma > 0


def _level_coords(u, v, L):
    Lf = jnp.float32(L)
    fu = jnp.clip(u * Lf - 0.5, 0.0, Lf - 1.0)
    fv = jnp.clip(v * Lf - 0.5, 0.0, Lf - 1.0)
    u0 = fu.astype(jnp.int32)
    v0 = fv.astype(jnp.int32)
    u1 = jnp.minimum(u0 + 1, L - 1)
    v1 = jnp.minimum(v0 + 1, L - 1)
    wu = fu - u0.astype(jnp.float32)
    wv = fv - v0.astype(jnp.float32)
    return u0, u1, v0, v1, wu, wv


def _lerp2(g00, g01, g10, g11, wu, wv):
    a = g00 + wu * (g01 - g00)
    b = g10 + wu * (g11 - g10)
    return a + wv * (b - a)


def _interleave(src_v, dst_v, vrows, L, iot, ccs):
    nj = L // _LANES

    def irow(r, carry):
        rbase = iot + r * L

        def ldj(j):
            idx = rbase + (j * _LANES)
            return [plsc.load_gather(src_v, [ccs[c], idx]) for c in range(_C)]

        g = ldj(0)
        for j in range(nj):
            gn = ldj(j + 1) if j + 1 < nj else None
            rows = rbase + (j * _LANES)
            for c in range(_C):
                plsc.store_scatter(dst_v, [rows, ccs[c]], g[c])
            g = gn
        return carry

    lax.fori_loop(0, vrows, irow, 0)


@functools.partial(
    pl.kernel,
    out_type=(jax.ShapeDtypeStruct((_B * 4 * _C,), jnp.float32),
              jax.ShapeDtypeStruct((_R2, _CP), jnp.float32),
              jax.ShapeDtypeStruct((_R3, _CP), jnp.float32)),
    mesh=plsc.VectorSubcoreMesh(core_axis_name="c", subcore_axis_name="s",
                                num_cores=_NC),
    compiler_params=pltpu.CompilerParams(needs_layout_passes=False,
                                         use_tc_tiling_on_sc=False),
    scratch_types=[
        pltpu.VMEM((6 * _C * _RES[0] * _RES[0],), jnp.float32),
        pltpu.VMEM((6 * _C * _RES[1] * _RES[1],), jnp.float32),
        pltpu.VMEM((2, _C, _T3CH), jnp.float32),
        pltpu.VMEM((_C, _T2CH), jnp.float32),
        pltpu.VMEM((2, _T3CH, _CP), jnp.float32),
        pltpu.VMEM((2, 3 * _CHUNK), jnp.float32),
        pltpu.VMEM((2, 4 * _CHUNK), jnp.int32),
        pltpu.VMEM((2, 4 * _CHUNK), jnp.int32),
        pltpu.VMEM((2, 4 * _CHUNK, _CP), jnp.float32),
        pltpu.VMEM((2, 4 * _CHUNK, _CP), jnp.float32),
        pltpu.VMEM((_C,), jnp.float32),
        pltpu.VMEM((2, _CHUNK * 4 * _C), jnp.float32),
        pltpu.SemaphoreType.DMA,
        pltpu.SemaphoreType.DMA,
        pltpu.SemaphoreType.DMA,
        pltpu.SemaphoreType.DMA,
        pltpu.SemaphoreType.DMA,
        pltpu.SemaphoreType.DMA,
        pltpu.SemaphoreType.DMA,
        pltpu.SemaphoreType.DMA,
        pltpu.SemaphoreType.DMA,
        pltpu.SemaphoreType.DMA,
        pltpu.SemaphoreType.DMA,
        pltpu.SemaphoreType.DMA,
    ],
)
def _encode_sc(inp_h, t0_h, t1_h, t2_h, t3_h, fail_h, out_h, t2r_h, t3r_h,
               t0_v, t1_v, pl_v, pl2_v, row_v, inp_v,
               idx2_v, idx3_v, taps2_v, taps3_v, fail_v, out_v,
               p1i_a, p1i_b, p1o_a, p1o_b, sin_a, sin_b,
               sg2_a, sg2_b, sg3_a, sg3_b, sout_a, sout_b):
    sid = lax.axis_index("s")
    wid = sid * _NC + lax.axis_index("c")
    iot = lax.iota(jnp.int32, _LANES)
    ccs = [jnp.full((_LANES,), c, jnp.int32) for c in range(_C)]
    p1i = (p1i_a, p1i_b)
    p1o = (p1o_a, p1o_b)
    sin = (sin_a, sin_b)
    sg = {2: (sg2_a, sg2_b), 3: (sg3_a, sg3_b)}
    sout = (sout_a, sout_b)
    tap_refs = {2: taps2_v, 3: taps3_v}
    idx_refs = {2: idx2_v, 3: idx3_v}
    src_refs = {2: t2r_h, 3: t3r_h}

    L3 = _RES[3]
    vrows3 = _T3CH // L3

    def p1_rb(ci):
        f = ci // _N3
        k = ci % _N3
        v0 = sid * (vrows3 * _N3) + k * vrows3
        return f * (L3 * L3) + v0 * L3

    def p1_fire_in(ci, par):
        f = ci // _N3
        k = ci % _N3
        v0 = sid * (vrows3 * _N3) + k * vrows3
        for c in range(_C):
            pb = (f * _C + c) * (L3 * L3) + v0 * L3
            pltpu.async_copy(t3_h.at[pl.ds(pb, _T3CH)], pl_v.at[par, c],
                             p1i[par])

    def p1_step(ci, par, first):
        rb = p1_rb(ci)
        for c in range(_C):
            pltpu.make_async_copy(t3_h.at[pl.ds(0, _T3CH)],
                                  pl_v.at[par, c], p1i[par]).wait()
        if not first:
            pltpu.make_async_copy(row_v.at[par],
                                  t3r_h.at[pl.ds(0, _T3CH)], p1o[par]).wait()
        _interleave(pl_v.at[par], row_v.at[par], vrows3, L3, iot, ccs)
        pltpu.async_copy(row_v.at[par], t3r_h.at[pl.ds(rb, _T3CH)], p1o[par])

    p1_fire_in(0, 0)
    p1_fire_in(1, 1)
    p1_step(0, 0, True)
    p1_fire_in(2, 0)
    p1_step(1, 1, True)
    p1_fire_in(3, 1)

    def p1_loop(kk, carry):
        ci = 2 + 2 * kk
        p1_step(ci, 0, False)
        p1_fire_in(ci + 2, 0)
        p1_step(ci + 1, 1, False)
        p1_fire_in(ci + 3, 1)
        return carry

    lax.fori_loop(0, (_NCH3 - 4) // 2, p1_loop, 0)
    p1_step(_NCH3 - 2, 0, False)
    p1_step(_NCH3 - 1, 1, False)
    pltpu.make_async_copy(row_v.at[0], t3r_h.at[pl.ds(0, _T3CH)], p1o[0]).wait()
    pltpu.make_async_copy(row_v.at[1], t3r_h.at[pl.ds(0, _T3CH)], p1o[1]).wait()

    L2 = _RES[2]
    vrows2 = _T2CH // L2

    def build2(f, carry):
        v0 = sid * vrows2
        for c in range(_C):
            pb = (f * _C + c) * (L2 * L2) + v0 * L2
            pltpu.sync_copy(t2_h.at[pl.ds(pb, _T2CH)], pl2_v.at[c])
        _interleave(pl2_v, row_v.at[0], vrows2, L2, iot, ccs)
        rb = f * (L2 * L2) + v0 * L2
        pltpu.sync_copy(row_v.at[0, pl.ds(0, _T2CH)], t2r_h.at[pl.ds(rb, _T2CH)])
        return carry

    lax.fori_loop(0, 6, build2, 0)

    pltpu.sync_copy(t0_h, t0_v)
    pltpu.sync_copy(t1_h, t1_v)
    pltpu.sync_copy(fail_h, fail_v)
    plsc.subcore_barrier()

    base0 = wid * _RPW
    fail_c = [plsc.load_gather(fail_v, [ccs[c]]) for c in range(_C)]
    rowm = [(iot + s * _LANES) * (4 * _C) for s in range(_NSUB)]
    trows = [[iot + (t * _CHUNK + s * _LANES) for t in range(4)]
             for s in range(_NSUB)]
    zero16 = jnp.zeros((_LANES,), jnp.int32)

    def fire_in(ci, par):
        pltpu.async_copy(inp_h.at[pl.ds((base0 + ci * _CHUNK) * 3, 3 * _CHUNK)],
                         inp_v.at[par], sin_a)

    def wait_in():
        pltpu.make_async_copy(inp_h.at[pl.ds(0, 3 * _CHUNK)],
                              inp_v.at[0], sin_a).wait()

    def fire_gathers(par, sems):
        pltpu.async_copy(t2r_h.at[idx2_v.at[par]], taps2_v.at[par], sems[0])
        pltpu.async_copy(t3r_h.at[idx3_v.at[par]], taps3_v.at[par], sems[1])

    def wait_gathers(par, sems):
        pltpu.make_async_copy(t2r_h.at[idx2_v.at[par]],
                              taps2_v.at[par], sems[0]).wait()
        pltpu.make_async_copy(t3r_h.at[idx3_v.at[par]],
                              taps3_v.at[par], sems[1]).wait()

    def fire_out(ci, par, sem):
        pltpu.async_copy(out_v.at[par],
                         out_h.at[pl.ds((base0 + ci * _CHUNK) * 4 * _C,
                                        _CHUNK * 4 * _C)], sem)

    def wait_out(sem):
        pltpu.make_async_copy(out_v.at[0],
                              out_h.at[pl.ds(0, _CHUNK * 4 * _C)], sem).wait()

    def phase_a(pv):
        subs = []
        for s in range(_NSUB):
            c0s = iot * 3 + (s * 3 * _LANES)
            x = plsc.load_gather(inp_v, [pv, c0s])
            y = plsc.load_gather(inp_v, [pv, c0s + 1])
            z = plsc.load_gather(inp_v, [pv, c0s + 2])
            face, u, v, ok = _dir_math(x, y, z)
            lv = [_level_coords(u, v, L) for L in _RES]
            for li, idx_r in ((2, idx2_v), (3, idx3_v)):
                L = _RES[li]
                u0, u1, v0, v1, wu, wv = lv[li]
                fb = face * (L * L)
                r0 = fb + v0 * L
                r1 = fb + v1 * L
                taps = (r0 + u0, r0 + u1, r1 + u0, r1 + u1)
                for t in range(4):
                    plsc.store_scatter(
                        idx_r, [pv, iot + (t * _CHUNK + s * _LANES)], taps[t])
            subs.append((face, ok, lv))
        return subs

    def l01(subs, pv):
        for s in range(_NSUB):
            face, ok, lv = subs[s]
            for li, tv in ((0, t0_v), (1, t1_v)):
                L = _RES[li]
                u0, u1, v0, v1, wu, wv = lv[li]
                fb = face * (_C * L * L)
                a00 = fb + v0 * L + u0
                a01 = fb + v0 * L + u1
                a10 = fb + v1 * L + u0
                a11 = fb + v1 * L + u1
                def ld01(c):
                    o = c * (L * L)
                    return (plsc.load_gather(tv, [a00 + o]),
                            plsc.load_gather(tv, [a01 + o]),
                            plsc.load_gather(tv, [a10 + o]),
                            plsc.load_gather(tv, [a11 + o]))

                g0, g1, g2 = ld01(0), ld01(1), ld01(2)
                for c in range(_C):
                    gn = ld01(c + 3) if c + 3 < _C else None
                    val = _lerp2(*g0, wu, wv)
                    val = jnp.where(ok, val, fail_c[c])
                    plsc.store_scatter(out_v,
                                       [pv, rowm[s] + (li * _C + c)], val)
                    g0, g1, g2 = g1, g2, gn
        return tuple(w for s in range(_NSUB)
                     for w in (subs[s][2][2][4], subs[s][2][2][5],
                               subs[s][2][3][4], subs[s][2][3][5],
                               jnp.where(subs[s][1], 1.0, 0.0)))

    def combine(w, pv):
        for s in range(_NSUB):
            wu2, wv2, wu3, wv3, okf = w[5 * s:5 * s + 5]
            ok = okf > 0.5
            for li, taps_r, wu, wv in ((2, taps2_v, wu2, wv2),
                                       (3, taps3_v, wu3, wv3)):
                def ldc(c, taps_r=taps_r):
                    return (plsc.load_gather(taps_r, [pv, trows[s][0], ccs[c]]),
                            plsc.load_gather(taps_r, [pv, trows[s][1], ccs[c]]),
                            plsc.load_gather(taps_r, [pv, trows[s][2], ccs[c]]),
                            plsc.load_gather(taps_r, [pv, trows[s][3], ccs[c]]))

                g0, g1, g2 = ldc(0), ldc(1), ldc(2)
                for c in range(_C):
                    gn = ldc(c + 3) if c + 3 < _C else None
                    val = _lerp2(*g0, wu, wv)
                    val = jnp.where(ok, val, fail_c[c])
                    plsc.store_scatter(out_v,
                                       [pv, rowm[s] + (li * _C + c)], val)
                    g0, g1, g2 = g1, g2, gn

    fire_in(0, 0)
    wait_in()
    subs0 = phase_a(zero16)
    fire_gathers(0, (sg2_a, sg3_a))
    w0 = l01(subs0, zero16)
    fire_in(1, 1)

    def loop(k, w):
        cur = k % 2
        nxt = 1 - cur
        pv_cur = zero16 + cur
        pv_nxt = zero16 + nxt

        @pl.when(k >= 1)
        def _():
            @pl.when(cur == 1)
            def _():
                wait_out(sout_a)
            @pl.when(cur == 0)
            def _():
                wait_out(sout_b)

        def prep(w_old):
            wait_in()
            subs = phase_a(pv_nxt)

            @pl.when(nxt == 0)
            def _():
                fire_gathers(0, (sg2_a, sg3_a))
            @pl.when(nxt == 1)
            def _():
                fire_gathers(1, (sg2_b, sg3_b))
            return l01(subs, pv_nxt)

        w_next = lax.cond(k < _NCHUNK - 1, prep, lambda w_old: w_old, w)

        @pl.when(cur == 0)
        def _():
            wait_gathers(0, (sg2_a, sg3_a))
            combine(w, zero16)
        @pl.when(cur == 1)
        def _():
            wait_gathers(1, (sg2_b, sg3_b))
            combine(w, zero16 + 1)

        @pl.when(cur == 0)
        def _():
            fire_out(k, 0, sout_a)
        @pl.when(cur == 1)
        def _():
            fire_out(k, 1, sout_b)

        @pl.when(k < _NCHUNK - 2)
        def _():
            fire_in(k + 2, cur)
        return w_next

    lax.fori_loop(0, _NCHUNK, loop, w0)
    wait_out(sout_b if (_NCHUNK - 1) % 2 == 1 else sout_a)


def kernel(inputs, params_0, params_1, params_2, params_3, fail_value):
    out, _, _ = _encode_sc(inputs.reshape(-1), params_0.reshape(-1),
                           params_1.reshape(-1), params_2.reshape(-1),
                           params_3.reshape(-1), fail_value)
    return out.reshape(_B, 4 * _C)


# speedup vs baseline: 1.0159x; 1.0159x over previous
import functools

import jax
import jax.numpy as jnp
from jax import lax
from jax.experimental import pallas as pl
from jax.experimental.pallas import tpu as pltpu
from jax.experimental.pallas import tpu_sc as plsc

_B = 262144
_C = 6
_RES = (8, 32, 128, 512)
_NC = 2
_NS = 16
_NW = _NC * _NS
_LANES = 16
_CHUNK = 64
_NSUB = _CHUNK // _LANES
_RPW = _B // _NW
_NCHUNK = _RPW // _CHUNK
_CP = 8
_R2 = 6 * _RES[2] * _RES[2]
_R3 = 6 * _RES[3] * _RES[3]
_T3CH = 1024
_T2CH = 1024
_N3 = (_RES[3] * _RES[3]) // (_NS * _T3CH)
_NCH3 = 6 * _N3


def _dir_math(x, y, z):
    ax, ay, az = jnp.abs(x), jnp.abs(y), jnp.abs(z)
    ma = jnp.maximum(jnp.maximum(ax, ay), az)
    is_x = (ax >= ay) & (ax >= az)
    is_y = (~is_x) & (ay >= az)
    face = jnp.where(
        is_x, jnp.where(x >= 0, 0, 1),
        jnp.where(is_y, jnp.where(y >= 0, 2, 3), jnp.where(z >= 0, 4, 5)),
    ).astype(jnp.int32)
    sc = jnp.where(is_x, jnp.where(x >= 0, -z, z),
                   jnp.where(is_y, x, jnp.where(z >= 0, x, -x)))
    tc = jnp.where(is_y, jnp.where(y >= 0, z, -z), -y)
    safe = jnp.where(ma > 0, ma, jnp.float32(1.0))
    u = 0.5 * (sc / safe + 1.0)
    v = 0.5 * (tc / safe + 1.0)
    return face, u, v, ma > 0


def _level_coords(u, v, L):
    Lf = jnp.float32(L)
    fu = jnp.clip(u * Lf - 0.5, 0.0, Lf - 1.0)
    fv = jnp.clip(v * Lf - 0.5, 0.0, Lf - 1.0)
    u0 = fu.astype(jnp.int32)
    v0 = fv.astype(jnp.int32)
    u1 = jnp.minimum(u0 + 1, L - 1)
    v1 = jnp.minimum(v0 + 1, L - 1)
    wu = fu - u0.astype(jnp.float32)
    wv = fv - v0.astype(jnp.float32)
    return u0, u1, v0, v1, wu, wv


def _lerp2(g00, g01, g10, g11, wu, wv):
    a = g00 + wu * (g01 - g00)
    b = g10 + wu * (g11 - g10)
    return a + wv * (b - a)


def _interleave(src_v, dst_v, iot, ccs):
    vrows, L = src_v.shape[1], src_v.shape[2]

    nj = L // _LANES

    def ldj(r, j):
        return [src_v[c, r, pl.ds(j * _LANES, _LANES)] for c in range(_C)]

    def irow(r, carry):
        rbase = iot + r * L
        g = ldj(r, 0)
        for j in range(nj):
            gn = ldj(r, j + 1) if j + 1 < nj else None
            rows = rbase + (j * _LANES)
            for c in range(_C):
                plsc.store_scatter(dst_v, [rows, ccs[c]], g[c])
            g = gn
        return carry

    lax.fori_loop(0, vrows, irow, 0)


@functools.partial(
    pl.kernel,
    out_type=(jax.ShapeDtypeStruct((_B * 4 * _C,), jnp.float32),
              jax.ShapeDtypeStruct((_R2, _CP), jnp.float32),
              jax.ShapeDtypeStruct((_R3, _CP), jnp.float32)),
    mesh=plsc.VectorSubcoreMesh(core_axis_name="c", subcore_axis_name="s",
                                num_cores=_NC),
    compiler_params=pltpu.CompilerParams(needs_layout_passes=False,
                                         use_tc_tiling_on_sc=False),
    scratch_types=[
        pltpu.VMEM((6 * _C * _RES[0] * _RES[0],), jnp.float32),
        pltpu.VMEM((6 * _C * _RES[1] * _RES[1],), jnp.float32),
        pltpu.VMEM((2, _C, _T3CH // _RES[3], _RES[3]), jnp.float32),
        pltpu.VMEM((_C, _T2CH // _RES[2], _RES[2]), jnp.float32),
        pltpu.VMEM((2, _T3CH, _CP), jnp.float32),
        pltpu.VMEM((2, 3 * _CHUNK), jnp.float32),
        pltpu.VMEM((2, 4 * _CHUNK), jnp.int32),
        pltpu.VMEM((2, 4 * _CHUNK), jnp.int32),
        pltpu.VMEM((2, 4 * _CHUNK, _CP), jnp.float32),
        pltpu.VMEM((2, 4 * _CHUNK, _CP), jnp.float32),
        pltpu.VMEM((_C,), jnp.float32),
        pltpu.VMEM((2, _CHUNK * 4 * _C), jnp.float32),
        pltpu.SemaphoreType.DMA,
        pltpu.SemaphoreType.DMA,
        pltpu.SemaphoreType.DMA,
        pltpu.SemaphoreType.DMA,
        pltpu.SemaphoreType.DMA,
        pltpu.SemaphoreType.DMA,
        pltpu.SemaphoreType.DMA,
        pltpu.SemaphoreType.DMA,
        pltpu.SemaphoreType.DMA,
        pltpu.SemaphoreType.DMA,
        pltpu.SemaphoreType.DMA,
        pltpu.SemaphoreType.DMA,
    ],
)
def _encode_sc(inp_h, t0_h, t1_h, t2_h, t3_h, fail_h, out_h, t2r_h, t3r_h,
               t0_v, t1_v, pl_v, pl2_v, row_v, inp_v,
               idx2_v, idx3_v, taps2_v, taps3_v, fail_v, out_v,
               p1i_a, p1i_b, p1o_a, p1o_b, sin_a, sin_b,
               sg2_a, sg2_b, sg3_a, sg3_b, sout_a, sout_b):
    sid = lax.axis_index("s")
    wid = sid * _NC + lax.axis_index("c")
    iot = lax.iota(jnp.int32, _LANES)
    ccs = [jnp.full((_LANES,), c, jnp.int32) for c in range(_C)]
    p1i = (p1i_a, p1i_b)
    p1o = (p1o_a, p1o_b)
    sin = (sin_a, sin_b)
    sg = {2: (sg2_a, sg2_b), 3: (sg3_a, sg3_b)}
    sout = (sout_a, sout_b)
    tap_refs = {2: taps2_v, 3: taps3_v}
    idx_refs = {2: idx2_v, 3: idx3_v}
    src_refs = {2: t2r_h, 3: t3r_h}

    L3 = _RES[3]
    vrows3 = _T3CH // L3

    def p1_src(ci):
        f = ci // _N3
        k = ci % _N3
        v0 = sid * (vrows3 * _N3) + k * vrows3
        return t3_h.at[f, :, pl.ds(v0, vrows3), :], f * (L3 * L3) + v0 * L3

    def p1_fire_in(ci, par):
        src, _ = p1_src(ci)
        pltpu.async_copy(src, pl_v.at[par], p1i[par])

    def p1_step(ci, par, first):
        src, rb = p1_src(ci)
        pltpu.make_async_copy(src, pl_v.at[par], p1i[par]).wait()
        if not first:
            pltpu.make_async_copy(row_v.at[par],
                                  t3r_h.at[pl.ds(0, _T3CH)], p1o[par]).wait()
        _interleave(pl_v.at[par], row_v.at[par], iot, ccs)
        pltpu.async_copy(row_v.at[par], t3r_h.at[pl.ds(rb, _T3CH)], p1o[par])

    p1_fire_in(0, 0)
    p1_fire_in(1, 1)
    p1_step(0, 0, True)
    p1_fire_in(2, 0)
    p1_step(1, 1, True)
    p1_fire_in(3, 1)

    def p1_loop(kk, carry):
        ci = 2 + 2 * kk
        p1_step(ci, 0, False)
        p1_fire_in(ci + 2, 0)
        p1_step(ci + 1, 1, False)
        p1_fire_in(ci + 3, 1)
        return carry

    lax.fori_loop(0, (_NCH3 - 4) // 2, p1_loop, 0)
    p1_step(_NCH3 - 2, 0, False)
    p1_step(_NCH3 - 1, 1, False)
    pltpu.make_async_copy(row_v.at[0], t3r_h.at[pl.ds(0, _T3CH)], p1o[0]).wait()
    pltpu.make_async_copy(row_v.at[1], t3r_h.at[pl.ds(0, _T3CH)], p1o[1]).wait()

    L2 = _RES[2]
    vrows2 = _T2CH // L2

    def build2(f, carry):
        v0 = sid * vrows2
        pltpu.sync_copy(t2_h.at[f, :, pl.ds(v0, vrows2), :], pl2_v)
        _interleave(pl2_v, row_v.at[0], iot, ccs)
        rb = f * (L2 * L2) + v0 * L2
        pltpu.sync_copy(row_v.at[0, pl.ds(0, _T2CH)], t2r_h.at[pl.ds(rb, _T2CH)])
        return carry

    lax.fori_loop(0, 6, build2, 0)

    pltpu.sync_copy(t0_h, t0_v)
    pltpu.sync_copy(t1_h, t1_v)
    pltpu.sync_copy(fail_h, fail_v)
    plsc.subcore_barrier()

    base0 = wid * _RPW
    fail_c = [plsc.load_gather(fail_v, [ccs[c]]) for c in range(_C)]
    rowm = [(iot + s * _LANES) * (4 * _C) for s in range(_NSUB)]
    trows = [[iot + (t * _CHUNK + s * _LANES) for t in range(4)]
             for s in range(_NSUB)]
    zero16 = jnp.zeros((_LANES,), jnp.int32)

    def fire_in(ci, par):
        pltpu.async_copy(inp_h.at[pl.ds((base0 + ci * _CHUNK) * 3, 3 * _CHUNK)],
                         inp_v.at[par], sin_a)

    def wait_in():
        pltpu.make_async_copy(inp_h.at[pl.ds(0, 3 * _CHUNK)],
                              inp_v.at[0], sin_a).wait()

    def fire_gathers(par, sems):
        pltpu.async_copy(t2r_h.at[idx2_v.at[par]], taps2_v.at[par], sems[0])
        pltpu.async_copy(t3r_h.at[idx3_v.at[par]], taps3_v.at[par], sems[1])

    def wait_gathers(par, sems):
        pltpu.make_async_copy(t2r_h.at[idx2_v.at[par]],
                              taps2_v.at[par], sems[0]).wait()
        pltpu.make_async_copy(t3r_h.at[idx3_v.at[par]],
                              taps3_v.at[par], sems[1]).wait()

    def fire_out(ci, par, sem):
        pltpu.async_copy(out_v.at[par],
                         out_h.at[pl.ds((base0 + ci * _CHUNK) * 4 * _C,
                                        _CHUNK * 4 * _C)], sem)

    def wait_out(sem):
        pltpu.make_async_copy(out_v.at[0],
                              out_h.at[pl.ds(0, _CHUNK * 4 * _C)], sem).wait()

    def phase_a(pv):
        def ldxyz(s):
            c0s = iot * 3 + (s * 3 * _LANES)
            return (plsc.load_gather(inp_v, [pv, c0s]),
                    plsc.load_gather(inp_v, [pv, c0s + 1]),
                    plsc.load_gather(inp_v, [pv, c0s + 2]))

        subs = []
        gxyz = ldxyz(0)
        for s in range(_NSUB):
            gn = ldxyz(s + 1) if s + 1 < _NSUB else None
            x, y, z = gxyz
            gxyz = gn
            face, u, v, ok = _dir_math(x, y, z)
            lv = [_level_coords(u, v, L) for L in _RES]
            for li, idx_r in ((2, idx2_v), (3, idx3_v)):
                L = _RES[li]
                u0, u1, v0, v1, wu, wv = lv[li]
                fb = face * (L * L)
                r0 = fb + v0 * L
                r1 = fb + v1 * L
                taps = (r0 + u0, r0 + u1, r1 + u0, r1 + u1)
                for t in range(4):
                    plsc.store_scatter(
                        idx_r, [pv, iot + (t * _CHUNK + s * _LANES)], taps[t])
            subs.append((face, ok, lv))
        return subs

    def l01(subs, pv):
        for s in range(_NSUB):
            face, ok, lv = subs[s]
            for li, tv in ((0, t0_v), (1, t1_v)):
                L = _RES[li]
                u0, u1, v0, v1, wu, wv = lv[li]
                fb = face * (_C * L * L)
                a00 = fb + v0 * L + u0
                a01 = fb + v0 * L + u1
                a10 = fb + v1 * L + u0
                a11 = fb + v1 * L + u1
                def ld01(c):
                    o = c * (L * L)
                    return (plsc.load_gather(tv, [a00 + o]),
                            plsc.load_gather(tv, [a01 + o]),
                            plsc.load_gather(tv, [a10 + o]),
                            plsc.load_gather(tv, [a11 + o]))

                g0, g1, g2 = ld01(0), ld01(1), ld01(2)
                for c in range(_C):
                    gn = ld01(c + 3) if c + 3 < _C else None
                    val = _lerp2(*g0, wu, wv)
                    val = jnp.where(ok, val, fail_c[c])
                    plsc.store_scatter(out_v,
                                       [pv, rowm[s] + (li * _C + c)], val)
                    g0, g1, g2 = g1, g2, gn
        return tuple(w for s in range(_NSUB)
                     for w in (subs[s][2][2][4], subs[s][2][2][5],
                               subs[s][2][3][4], subs[s][2][3][5],
                               jnp.where(subs[s][1], 1.0, 0.0)))

    def combine(w, pv):
        for s in range(_NSUB):
            wu2, wv2, wu3, wv3, okf = w[5 * s:5 * s + 5]
            ok = okf > 0.5
            for li, taps_r, wu, wv in ((2, taps2_v, wu2, wv2),
                                       (3, taps3_v, wu3, wv3)):
                def ldc(c, taps_r=taps_r):
                    return (plsc.load_gather(taps_r, [pv, trows[s][0], ccs[c]]),
                            plsc.load_gather(taps_r, [pv, trows[s][1], ccs[c]]),
                            plsc.load_gather(taps_r, [pv, trows[s][2], ccs[c]]),
                            plsc.load_gather(taps_r, [pv, trows[s][3], ccs[c]]))

                g0, g1, g2 = ldc(0), ldc(1), ldc(2)
                for c in range(_C):
                    gn = ldc(c + 3) if c + 3 < _C else None
                    val = _lerp2(*g0, wu, wv)
                    val = jnp.where(ok, val, fail_c[c])
                    plsc.store_scatter(out_v,
                                       [pv, rowm[s] + (li * _C + c)], val)
                    g0, g1, g2 = g1, g2, gn

    fire_in(0, 0)
    wait_in()
    subs0 = phase_a(zero16)
    fire_gathers(0, (sg2_a, sg3_a))
    w0 = l01(subs0, zero16)
    fire_in(1, 1)

    def loop(k, w):
        cur = k % 2
        nxt = 1 - cur
        pv_cur = zero16 + cur
        pv_nxt = zero16 + nxt

        @pl.when(k >= 1)
        def _():
            @pl.when(cur == 1)
            def _():
                wait_out(sout_a)
            @pl.when(cur == 0)
            def _():
                wait_out(sout_b)

        def prep(w_old):
            wait_in()
            subs = phase_a(pv_nxt)

            @pl.when(nxt == 0)
            def _():
                fire_gathers(0, (sg2_a, sg3_a))
            @pl.when(nxt == 1)
            def _():
                fire_gathers(1, (sg2_b, sg3_b))
            return l01(subs, pv_nxt)

        w_next = lax.cond(k < _NCHUNK - 1, prep, lambda w_old: w_old, w)

        @pl.when(cur == 0)
        def _():
            wait_gathers(0, (sg2_a, sg3_a))
            combine(w, zero16)
        @pl.when(cur == 1)
        def _():
            wait_gathers(1, (sg2_b, sg3_b))
            combine(w, zero16 + 1)

        @pl.when(cur == 0)
        def _():
            fire_out(k, 0, sout_a)
        @pl.when(cur == 1)
        def _():
            fire_out(k, 1, sout_b)

        @pl.when(k < _NCHUNK - 2)
        def _():
            fire_in(k + 2, cur)
        return w_next

    lax.fori_loop(0, _NCHUNK, loop, w0)
    wait_out(sout_b if (_NCHUNK - 1) % 2 == 1 else sout_a)


def kernel(inputs, params_0, params_1, params_2, params_3, fail_value):
    out, _, _ = _encode_sc(inputs.reshape(-1), params_0.reshape(-1),
                           params_1.reshape(-1), params_2, params_3,
                           fail_value)
    return out.reshape(_B, 4 * _C)


# speedup vs baseline: 1.0296x; 1.0134x over previous
import functools

import jax
import jax.numpy as jnp
from jax import lax
from jax.experimental import pallas as pl
from jax.experimental.pallas import tpu as pltpu
from jax.experimental.pallas import tpu_sc as plsc

_B = 262144
_C = 6
_RES = (8, 32, 128, 512)
_NC = 2
_NS = 16
_NW = _NC * _NS
_LANES = 16
_CHUNK = 64
_NSUB = _CHUNK // _LANES
_RPW = _B // _NW
_NCHUNK = _RPW // _CHUNK
_CP = 8
_R2 = 6 * _RES[2] * _RES[2]
_R3 = 6 * _RES[3] * _RES[3]
_T3CH = 1024
_T2CH = 1024
_N3 = (_RES[3] * _RES[3]) // (_NS * _T3CH)
_NCH3 = 6 * _N3


def _dir_math(x, y, z):
    ax, ay, az = jnp.abs(x), jnp.abs(y), jnp.abs(z)
    ma = jnp.maximum(jnp.maximum(ax, ay), az)
    is_x = (ax >= ay) & (ax >= az)
    is_y = (~is_x) & (ay >= az)
    face = jnp.where(
        is_x, jnp.where(x >= 0, 0, 1),
        jnp.where(is_y, jnp.where(y >= 0, 2, 3), jnp.where(z >= 0, 4, 5)),
    ).astype(jnp.int32)
    sc = jnp.where(is_x, jnp.where(x >= 0, -z, z),
                   jnp.where(is_y, x, jnp.where(z >= 0, x, -x)))
    tc = jnp.where(is_y, jnp.where(y >= 0, z, -z), -y)
    safe = jnp.where(ma > 0, ma, jnp.float32(1.0))
    u = 0.5 * (sc / safe + 1.0)
    v = 0.5 * (tc / safe + 1.0)
    return face, u, v, ma > 0


def _level_coords(u, v, L):
    Lf = jnp.float32(L)
    fu = jnp.clip(u * Lf - 0.5, 0.0, Lf - 1.0)
    fv = jnp.clip(v * Lf - 0.5, 0.0, Lf - 1.0)
    u0 = fu.astype(jnp.int32)
    v0 = fv.astype(jnp.int32)
    u1 = jnp.minimum(u0 + 1, L - 1)
    v1 = jnp.minimum(v0 + 1, L - 1)
    wu = fu - u0.astype(jnp.float32)
    wv = fv - v0.astype(jnp.float32)
    return u0, u1, v0, v1, wu, wv


def _lerp2(g00, g01, g10, g11, wu, wv):
    a = g00 + wu * (g01 - g00)
    b = g10 + wu * (g11 - g10)
    return a + wv * (b - a)


def _interleave(src_v, dst_v, iot, ccs):
    vrows, L = src_v.shape[1], src_v.shape[2]

    nj = L // _LANES

    def ldj(r, j):
        return [src_v[c, r, pl.ds(j * _LANES, _LANES)] for c in range(_C)]

    def irow(r, carry):
        rbase = iot + r * L
        g = ldj(r, 0)
        for j in range(nj):
            gn = ldj(r, j + 1) if j + 1 < nj else None
            rows = rbase + (j * _LANES)
            for c in range(_C):
                plsc.store_scatter(dst_v, [rows, ccs[c]], g[c])
            g = gn
        return carry

    lax.fori_loop(0, vrows, irow, 0)


@functools.partial(
    pl.kernel,
    out_type=(jax.ShapeDtypeStruct((_B * 4 * _C,), jnp.float32),
              jax.ShapeDtypeStruct((_R2, _CP), jnp.float32),
              jax.ShapeDtypeStruct((_R3, _CP), jnp.float32)),
    mesh=plsc.VectorSubcoreMesh(core_axis_name="c", subcore_axis_name="s",
                                num_cores=_NC),
    compiler_params=pltpu.CompilerParams(needs_layout_passes=False,
                                         use_tc_tiling_on_sc=False),
    scratch_types=[
        pltpu.VMEM((6 * _C * _RES[0] * _RES[0],), jnp.float32),
        pltpu.VMEM((6 * _C * _RES[1] * _RES[1],), jnp.float32),
        pltpu.VMEM((2, _C, _T3CH // _RES[3], _RES[3]), jnp.float32),
        pltpu.VMEM((_C, _T2CH // _RES[2], _RES[2]), jnp.float32),
        pltpu.VMEM((2, _T3CH, _CP), jnp.float32),
        pltpu.VMEM((2, 3 * _CHUNK), jnp.float32),
        pltpu.VMEM((2, 4 * _CHUNK), jnp.int32),
        pltpu.VMEM((2, 4 * _CHUNK), jnp.int32),
        pltpu.VMEM((2, 4 * _CHUNK, _CP), jnp.float32),
        pltpu.VMEM((2, 4 * _CHUNK, _CP), jnp.float32),
        pltpu.VMEM((_C,), jnp.float32),
        pltpu.VMEM((2, _CHUNK * 4 * _C), jnp.float32),
        pltpu.SemaphoreType.DMA,
        pltpu.SemaphoreType.DMA,
        pltpu.SemaphoreType.DMA,
        pltpu.SemaphoreType.DMA,
        pltpu.SemaphoreType.DMA,
        pltpu.SemaphoreType.DMA,
        pltpu.SemaphoreType.DMA,
        pltpu.SemaphoreType.DMA,
        pltpu.SemaphoreType.DMA,
        pltpu.SemaphoreType.DMA,
        pltpu.SemaphoreType.DMA,
        pltpu.SemaphoreType.DMA,
    ],
)
def _encode_sc(inp_h, t0_h, t1_h, t2_h, t3_h, fail_h, out_h, t2r_h, t3r_h,
               t0_v, t1_v, pl_v, pl2_v, row_v, inp_v,
               idx2_v, idx3_v, taps2_v, taps3_v, fail_v, out_v,
               p1i_a, p1i_b, p1o_a, p1o_b, sin_a, sin_b,
               sg2_a, sg2_b, sg3_a, sg3_b, sout_a, sout_b):
    sid = lax.axis_index("s")
    wid = sid * _NC + lax.axis_index("c")
    iot = lax.iota(jnp.int32, _LANES)
    ccs = [jnp.full((_LANES,), c, jnp.int32) for c in range(_C)]
    p1i = (p1i_a, p1i_b)
    p1o = (p1o_a, p1o_b)
    sin = (sin_a, sin_b)
    sg = {2: (sg2_a, sg2_b), 3: (sg3_a, sg3_b)}
    sout = (sout_a, sout_b)
    tap_refs = {2: taps2_v, 3: taps3_v}
    idx_refs = {2: idx2_v, 3: idx3_v}
    src_refs = {2: t2r_h, 3: t3r_h}

    L3 = _RES[3]
    vrows3 = _T3CH // L3

    def p1_src(ci):
        f = ci // _N3
        k = ci % _N3
        v0 = sid * (vrows3 * _N3) + k * vrows3
        return t3_h.at[f, :, pl.ds(v0, vrows3), :], f * (L3 * L3) + v0 * L3

    def p1_fire_in(ci, par):
        src, _ = p1_src(ci)
        pltpu.async_copy(src, pl_v.at[par], p1i[par])

    def p1_step(ci, par, first):
        src, rb = p1_src(ci)
        pltpu.make_async_copy(src, pl_v.at[par], p1i[par]).wait()
        if not first:
            pltpu.make_async_copy(row_v.at[par],
                                  t3r_h.at[pl.ds(0, _T3CH)], p1o[par]).wait()
        _interleave(pl_v.at[par], row_v.at[par], iot, ccs)
        pltpu.async_copy(row_v.at[par], t3r_h.at[pl.ds(rb, _T3CH)], p1o[par])

    p1_fire_in(0, 0)
    p1_fire_in(1, 1)
    p1_step(0, 0, True)
    p1_fire_in(2, 0)
    p1_step(1, 1, True)
    p1_fire_in(3, 1)

    def p1_loop(kk, carry):
        ci = 2 + 2 * kk
        p1_step(ci, 0, False)
        p1_fire_in(ci + 2, 0)
        p1_step(ci + 1, 1, False)
        p1_fire_in(ci + 3, 1)
        return carry

    lax.fori_loop(0, (_NCH3 - 4) // 2, p1_loop, 0)
    p1_step(_NCH3 - 2, 0, False)
    p1_step(_NCH3 - 1, 1, False)
    pltpu.make_async_copy(row_v.at[0], t3r_h.at[pl.ds(0, _T3CH)], p1o[0]).wait()
    pltpu.make_async_copy(row_v.at[1], t3r_h.at[pl.ds(0, _T3CH)], p1o[1]).wait()

    L2 = _RES[2]
    vrows2 = _T2CH // L2

    def build2(f, carry):
        v0 = sid * vrows2
        pltpu.sync_copy(t2_h.at[f, :, pl.ds(v0, vrows2), :], pl2_v)
        _interleave(pl2_v, row_v.at[0], iot, ccs)
        rb = f * (L2 * L2) + v0 * L2
        pltpu.sync_copy(row_v.at[0, pl.ds(0, _T2CH)], t2r_h.at[pl.ds(rb, _T2CH)])
        return carry

    lax.fori_loop(0, 6, build2, 0)

    pltpu.sync_copy(t0_h, t0_v)
    pltpu.sync_copy(t1_h, t1_v)
    pltpu.sync_copy(fail_h, fail_v)
    plsc.subcore_barrier()

    base0 = wid * _RPW
    fail_c = [plsc.load_gather(fail_v, [ccs[c]]) for c in range(_C)]
    rowm = [(iot + s * _LANES) * (4 * _C) for s in range(_NSUB)]
    trows = [[iot + (t * _CHUNK + s * _LANES) for t in range(4)]
             for s in range(_NSUB)]
    zero16 = jnp.zeros((_LANES,), jnp.int32)

    def fire_in(ci, par):
        pltpu.async_copy(inp_h.at[pl.ds((base0 + ci * _CHUNK) * 3, 3 * _CHUNK)],
                         inp_v.at[par], sin_a)

    def wait_in():
        pltpu.make_async_copy(inp_h.at[pl.ds(0, 3 * _CHUNK)],
                              inp_v.at[0], sin_a).wait()

    def fire_gathers(par, sems):
        pltpu.async_copy(t2r_h.at[idx2_v.at[par]], taps2_v.at[par], sems[0])
        pltpu.async_copy(t3r_h.at[idx3_v.at[par]], taps3_v.at[par], sems[1])

    def wait_gathers(par, sems):
        pltpu.make_async_copy(t2r_h.at[idx2_v.at[par]],
                              taps2_v.at[par], sems[0]).wait()
        pltpu.make_async_copy(t3r_h.at[idx3_v.at[par]],
                              taps3_v.at[par], sems[1]).wait()

    def fire_out(ci, par, sem):
        pltpu.async_copy(out_v.at[par],
                         out_h.at[pl.ds((base0 + ci * _CHUNK) * 4 * _C,
                                        _CHUNK * 4 * _C)], sem)

    def wait_out(sem):
        pltpu.make_async_copy(out_v.at[0],
                              out_h.at[pl.ds(0, _CHUNK * 4 * _C)], sem).wait()

    def phase_a(pv):
        subs = []
        for s in range(_NSUB):
            c0s = iot * 3 + (s * 3 * _LANES)
            x = plsc.load_gather(inp_v, [pv, c0s])
            y = plsc.load_gather(inp_v, [pv, c0s + 1])
            z = plsc.load_gather(inp_v, [pv, c0s + 2])
            face, u, v, ok = _dir_math(x, y, z)
            lv = [_level_coords(u, v, L) for L in _RES]
            for li, idx_r in ((2, idx2_v), (3, idx3_v)):
                L = _RES[li]
                u0, u1, v0, v1, wu, wv = lv[li]
                fb = face * (L * L)
                r0 = fb + v0 * L
                r1 = fb + v1 * L
                taps = (r0 + u0, r0 + u1, r1 + u0, r1 + u1)
                for t in range(4):
                    plsc.store_scatter(
                        idx_r, [pv, iot + (t * _CHUNK + s * _LANES)], taps[t])
            subs.append((face, ok, lv))
        return subs

    def l01(subs, pv):
        for s in range(_NSUB):
            face, ok, lv = subs[s]
            for li, tv in ((0, t0_v), (1, t1_v)):
                L = _RES[li]
                u0, u1, v0, v1, wu, wv = lv[li]
                fb = face * (_C * L * L)
                a00 = fb + v0 * L + u0
                a01 = fb + v0 * L + u1
                a10 = fb + v1 * L + u0
                a11 = fb + v1 * L + u1
                def ld01(c):
                    o = c * (L * L)
                    return (plsc.load_gather(tv, [a00 + o]),
                            plsc.load_gather(tv, [a01 + o]),
                            plsc.load_gather(tv, [a10 + o]),
                            plsc.load_gather(tv, [a11 + o]))

                g0, g1, g2 = ld01(0), ld01(1), ld01(2)
                for c in range(_C):
                    gn = ld01(c + 3) if c + 3 < _C else None
                    val = _lerp2(*g0, wu, wv)
                    val = jnp.where(ok, val, fail_c[c])
                    plsc.store_scatter(out_v,
                                       [pv, rowm[s] + (li * _C + c)], val)
                    g0, g1, g2 = g1, g2, gn
        return tuple(w for s in range(_NSUB)
                     for w in (subs[s][2][2][4], subs[s][2][2][5],
                               subs[s][2][3][4], subs[s][2][3][5],
                               jnp.where(subs[s][1], 1.0, 0.0)))

    def combine(w, pv):
        for s in range(_NSUB):
            wu2, wv2, wu3, wv3, okf = w[5 * s:5 * s + 5]
            ok = okf > 0.5
            for li, taps_r, wu, wv in ((2, taps2_v, wu2, wv2),
                                       (3, taps3_v, wu3, wv3)):
                def ldc(c, taps_r=taps_r):
                    return (plsc.load_gather(taps_r, [pv, trows[s][0], ccs[c]]),
                            plsc.load_gather(taps_r, [pv, trows[s][1], ccs[c]]),
                            plsc.load_gather(taps_r, [pv, trows[s][2], ccs[c]]),
                            plsc.load_gather(taps_r, [pv, trows[s][3], ccs[c]]))

                g0, g1, g2 = ldc(0), ldc(1), ldc(2)
                for c in range(_C):
                    gn = ldc(c + 3) if c + 3 < _C else None
                    val = _lerp2(*g0, wu, wv)
                    val = jnp.where(ok, val, fail_c[c])
                    plsc.store_scatter(out_v,
                                       [pv, rowm[s] + (li * _C + c)], val)
                    g0, g1, g2 = g1, g2, gn

    fire_in(0, 0)
    wait_in()
    subs0 = phase_a(zero16)
    fire_gathers(0, (sg2_a, sg3_a))
    w0 = l01(subs0, zero16)
    fire_in(1, 1)

    def loop(k, w):
        cur = k % 2
        nxt = 1 - cur
        pv_cur = zero16 + cur
        pv_nxt = zero16 + nxt

        @pl.when(k >= 1)
        def _():
            @pl.when(cur == 1)
            def _():
                wait_out(sout_a)
            @pl.when(cur == 0)
            def _():
                wait_out(sout_b)

        def prep(w_old):
            wait_in()
            subs = phase_a(pv_nxt)

            @pl.when(nxt == 0)
            def _():
                fire_gathers(0, (sg2_a, sg3_a))
            @pl.when(nxt == 1)
            def _():
                fire_gathers(1, (sg2_b, sg3_b))
            return l01(subs, pv_nxt)

        w_next = lax.cond(k < _NCHUNK - 1, prep, lambda w_old: w_old, w)

        @pl.when(cur == 0)
        def _():
            wait_gathers(0, (sg2_a, sg3_a))
            combine(w, zero16)
        @pl.when(cur == 1)
        def _():
            wait_gathers(1, (sg2_b, sg3_b))
            combine(w, zero16 + 1)

        @pl.when(cur == 0)
        def _():
            fire_out(k, 0, sout_a)
        @pl.when(cur == 1)
        def _():
            fire_out(k, 1, sout_b)

        @pl.when(k < _NCHUNK - 2)
        def _():
            fire_in(k + 2, cur)
        return w_next

    lax.fori_loop(0, _NCHUNK, loop, w0)
    wait_out(sout_b if (_NCHUNK - 1) % 2 == 1 else sout_a)


def kernel(inputs, params_0, params_1, params_2, params_3, fail_value):
    out, _, _ = _encode_sc(inputs.reshape(-1), params_0.reshape(-1),
                           params_1.reshape(-1), params_2, params_3,
                           fail_value)
    return out.reshape(_B, 4 * _C)


# speedup vs baseline: 1.0562x; 1.0259x over previous
import functools

import jax
import jax.numpy as jnp
from jax import lax
from jax.experimental import pallas as pl
from jax.experimental.pallas import tpu as pltpu
from jax.experimental.pallas import tpu_sc as plsc

_B = 262144
_C = 6
_RES = (8, 32, 128, 512)
_NC = 2
_NS = 16
_NW = _NC * _NS
_LANES = 16
_CHUNK = 64
_NSUB = _CHUNK // _LANES
_RPW = _B // _NW
_NCHUNK = _RPW // _CHUNK
_CP = 8
_R2 = 6 * _RES[2] * _RES[2]
_R3 = 6 * _RES[3] * _RES[3]
_T3CH = 2048
_T2CH = 1024
_N3 = (_RES[3] * _RES[3]) // (_NS * _T3CH)
_NCH3 = 6 * _N3


def _dir_math(x, y, z):
    ax, ay, az = jnp.abs(x), jnp.abs(y), jnp.abs(z)
    ma = jnp.maximum(jnp.maximum(ax, ay), az)
    is_x = (ax >= ay) & (ax >= az)
    is_y = (~is_x) & (ay >= az)
    face = jnp.where(
        is_x, jnp.where(x >= 0, 0, 1),
        jnp.where(is_y, jnp.where(y >= 0, 2, 3), jnp.where(z >= 0, 4, 5)),
    ).astype(jnp.int32)
    sc = jnp.where(is_x, jnp.where(x >= 0, -z, z),
                   jnp.where(is_y, x, jnp.where(z >= 0, x, -x)))
    tc = jnp.where(is_y, jnp.where(y >= 0, z, -z), -y)
    safe = jnp.where(ma > 0, ma, jnp.float32(1.0))
    u = 0.5 * (sc / safe + 1.0)
    v = 0.5 * (tc / safe + 1.0)
    return face, u, v, ma > 0


def _level_coords(u, v, L):
    Lf = jnp.float32(L)
    fu = jnp.clip(u * Lf - 0.5, 0.0, Lf - 1.0)
    fv = jnp.clip(v * Lf - 0.5, 0.0, Lf - 1.0)
    u0 = fu.astype(jnp.int32)
    v0 = fv.astype(jnp.int32)
    u1 = jnp.minimum(u0 + 1, L - 1)
    v1 = jnp.minimum(v0 + 1, L - 1)
    wu = fu - u0.astype(jnp.float32)
    wv = fv - v0.astype(jnp.float32)
    return u0, u1, v0, v1, wu, wv


def _lerp2(g00, g01, g10, g11, wu, wv):
    a = g00 + wu * (g01 - g00)
    b = g10 + wu * (g11 - g10)
    return a + wv * (b - a)


def _interleave(src_v, dst_v, iot, ccs):
    vrows, L = src_v.shape[1], src_v.shape[2]

    nj = L // _LANES

    def ldj(r, j):
        return [src_v[c, r, pl.ds(j * _LANES, _LANES)] for c in range(_C)]

    def irow(r, carry):
        rbase = iot + r * L
        g = ldj(r, 0)
        for j in range(nj):
            gn = ldj(r, j + 1) if j + 1 < nj else None
            rows = rbase + (j * _LANES)
            for c in range(_C):
                plsc.store_scatter(dst_v, [rows, ccs[c]], g[c])
            g = gn
        return carry

    lax.fori_loop(0, vrows, irow, 0)


@functools.partial(
    pl.kernel,
    out_type=(jax.ShapeDtypeStruct((_B * 4 * _C,), jnp.float32),
              jax.ShapeDtypeStruct((_R2, _CP), jnp.float32),
              jax.ShapeDtypeStruct((_R3, _CP), jnp.float32)),
    mesh=plsc.VectorSubcoreMesh(core_axis_name="c", subcore_axis_name="s",
                                num_cores=_NC),
    compiler_params=pltpu.CompilerParams(needs_layout_passes=False,
                                         use_tc_tiling_on_sc=False),
    scratch_types=[
        pltpu.VMEM((6 * _C * _RES[0] * _RES[0],), jnp.float32),
        pltpu.VMEM((6 * _C * _RES[1] * _RES[1],), jnp.float32),
        pltpu.VMEM((2, _C, _T3CH // _RES[3], _RES[3]), jnp.float32),
        pltpu.VMEM((_C, _T2CH // _RES[2], _RES[2]), jnp.float32),
        pltpu.VMEM((2, _T3CH, _CP), jnp.float32),
        pltpu.VMEM((2, 3 * _CHUNK), jnp.float32),
        pltpu.VMEM((2, 4 * _CHUNK), jnp.int32),
        pltpu.VMEM((2, 4 * _CHUNK), jnp.int32),
        pltpu.VMEM((2, 4 * _CHUNK, _CP), jnp.float32),
        pltpu.VMEM((2, 4 * _CHUNK, _CP), jnp.float32),
        pltpu.VMEM((_C,), jnp.float32),
        pltpu.VMEM((2, _CHUNK * 4 * _C), jnp.float32),
        pltpu.SemaphoreType.DMA,
        pltpu.SemaphoreType.DMA,
        pltpu.SemaphoreType.DMA,
        pltpu.SemaphoreType.DMA,
        pltpu.SemaphoreType.DMA,
        pltpu.SemaphoreType.DMA,
        pltpu.SemaphoreType.DMA,
        pltpu.SemaphoreType.DMA,
        pltpu.SemaphoreType.DMA,
        pltpu.SemaphoreType.DMA,
        pltpu.SemaphoreType.DMA,
        pltpu.SemaphoreType.DMA,
    ],
)
def _encode_sc(inp_h, t0_h, t1_h, t2_h, t3_h, fail_h, out_h, t2r_h, t3r_h,
               t0_v, t1_v, pl_v, pl2_v, row_v, inp_v,
               idx2_v, idx3_v, taps2_v, taps3_v, fail_v, out_v,
               p1i_a, p1i_b, p1o_a, p1o_b, sin_a, sin_b,
               sg2_a, sg2_b, sg3_a, sg3_b, sout_a, sout_b):
    sid = lax.axis_index("s")
    wid = sid * _NC + lax.axis_index("c")
    iot = lax.iota(jnp.int32, _LANES)
    ccs = [jnp.full((_LANES,), c, jnp.int32) for c in range(_C)]
    p1i = (p1i_a, p1i_b)
    p1o = (p1o_a, p1o_b)
    sin = (sin_a, sin_b)
    sg = {2: (sg2_a, sg2_b), 3: (sg3_a, sg3_b)}
    sout = (sout_a, sout_b)
    tap_refs = {2: taps2_v, 3: taps3_v}
    idx_refs = {2: idx2_v, 3: idx3_v}
    src_refs = {2: t2r_h, 3: t3r_h}

    L3 = _RES[3]
    vrows3 = _T3CH // L3

    def p1_src(ci):
        f = ci // _N3
        k = ci % _N3
        v0 = sid * (vrows3 * _N3) + k * vrows3
        return t3_h.at[f, :, pl.ds(v0, vrows3), :], f * (L3 * L3) + v0 * L3

    def p1_fire_in(ci, par):
        src, _ = p1_src(ci)
        pltpu.async_copy(src, pl_v.at[par], p1i[par])

    def p1_step(ci, par, first):
        src, rb = p1_src(ci)
        pltpu.make_async_copy(src, pl_v.at[par], p1i[par]).wait()
        if not first:
            pltpu.make_async_copy(row_v.at[par],
                                  t3r_h.at[pl.ds(0, _T3CH)], p1o[par]).wait()
        _interleave(pl_v.at[par], row_v.at[par], iot, ccs)
        pltpu.async_copy(row_v.at[par], t3r_h.at[pl.ds(rb, _T3CH)], p1o[par])

    p1_fire_in(0, 0)
    p1_fire_in(1, 1)
    p1_step(0, 0, True)
    p1_fire_in(2, 0)
    p1_step(1, 1, True)
    p1_fire_in(3, 1)

    def p1_loop(kk, carry):
        ci = 2 + 2 * kk
        p1_step(ci, 0, False)
        p1_fire_in(ci + 2, 0)
        p1_step(ci + 1, 1, False)
        p1_fire_in(ci + 3, 1)
        return carry

    lax.fori_loop(0, (_NCH3 - 4) // 2, p1_loop, 0)
    p1_step(_NCH3 - 2, 0, False)
    p1_step(_NCH3 - 1, 1, False)
    pltpu.make_async_copy(row_v.at[0], t3r_h.at[pl.ds(0, _T3CH)], p1o[0]).wait()
    pltpu.make_async_copy(row_v.at[1], t3r_h.at[pl.ds(0, _T3CH)], p1o[1]).wait()

    L2 = _RES[2]
    vrows2 = _T2CH // L2

    def build2(f, carry):
        v0 = sid * vrows2
        pltpu.sync_copy(t2_h.at[f, :, pl.ds(v0, vrows2), :], pl2_v)
        _interleave(pl2_v, row_v.at[0], iot, ccs)
        rb = f * (L2 * L2) + v0 * L2
        pltpu.sync_copy(row_v.at[0, pl.ds(0, _T2CH)], t2r_h.at[pl.ds(rb, _T2CH)])
        return carry

    lax.fori_loop(0, 6, build2, 0)

    pltpu.sync_copy(t0_h, t0_v)
    pltpu.sync_copy(t1_h, t1_v)
    pltpu.sync_copy(fail_h, fail_v)
    plsc.subcore_barrier()

    base0 = wid * _RPW
    fail_c = [plsc.load_gather(fail_v, [ccs[c]]) for c in range(_C)]
    rowm = [(iot + s * _LANES) * (4 * _C) for s in range(_NSUB)]
    trows = [[iot + (t * _CHUNK + s * _LANES) for t in range(4)]
             for s in range(_NSUB)]
    zero16 = jnp.zeros((_LANES,), jnp.int32)

    def fire_in(ci, par):
        pltpu.async_copy(inp_h.at[pl.ds((base0 + ci * _CHUNK) * 3, 3 * _CHUNK)],
                         inp_v.at[par], sin_a)

    def wait_in():
        pltpu.make_async_copy(inp_h.at[pl.ds(0, 3 * _CHUNK)],
                              inp_v.at[0], sin_a).wait()

    def fire_gathers(par, sems):
        pltpu.async_copy(t2r_h.at[idx2_v.at[par]], taps2_v.at[par], sems[0])
        pltpu.async_copy(t3r_h.at[idx3_v.at[par]], taps3_v.at[par], sems[1])

    def wait_gathers(par, sems):
        pltpu.make_async_copy(t2r_h.at[idx2_v.at[par]],
                              taps2_v.at[par], sems[0]).wait()
        pltpu.make_async_copy(t3r_h.at[idx3_v.at[par]],
                              taps3_v.at[par], sems[1]).wait()

    def fire_out(ci, par, sem):
        pltpu.async_copy(out_v.at[par],
                         out_h.at[pl.ds((base0 + ci * _CHUNK) * 4 * _C,
                                        _CHUNK * 4 * _C)], sem)

    def wait_out(sem):
        pltpu.make_async_copy(out_v.at[0],
                              out_h.at[pl.ds(0, _CHUNK * 4 * _C)], sem).wait()

    def phase_a(pv):
        subs = []
        for s in range(_NSUB):
            c0s = iot * 3 + (s * 3 * _LANES)
            x = plsc.load_gather(inp_v, [pv, c0s])
            y = plsc.load_gather(inp_v, [pv, c0s + 1])
            z = plsc.load_gather(inp_v, [pv, c0s + 2])
            face, u, v, ok = _dir_math(x, y, z)
            lv = [_level_coords(u, v, L) for L in _RES]
            for li, idx_r in ((2, idx2_v), (3, idx3_v)):
                L = _RES[li]
                u0, u1, v0, v1, wu, wv = lv[li]
                fb = face * (L * L)
                r0 = fb + v0 * L
                r1 = fb + v1 * L
                taps = (r0 + u0, r0 + u1, r1 + u0, r1 + u1)
                for t in range(4):
                    plsc.store_scatter(
                        idx_r, [pv, iot + (t * _CHUNK + s * _LANES)], taps[t])
            subs.append((face, ok, lv))
        return subs

    def l01(subs, pv):
        for s in range(_NSUB):
            face, ok, lv = subs[s]
            for li, tv in ((0, t0_v), (1, t1_v)):
                L = _RES[li]
                u0, u1, v0, v1, wu, wv = lv[li]
                fb = face * (_C * L * L)
                a00 = fb + v0 * L + u0
                a01 = fb + v0 * L + u1
                a10 = fb + v1 * L + u0
                a11 = fb + v1 * L + u1
                def ld01(c):
                    o = c * (L * L)
                    return (plsc.load_gather(tv, [a00 + o]),
                            plsc.load_gather(tv, [a01 + o]),
                            plsc.load_gather(tv, [a10 + o]),
                            plsc.load_gather(tv, [a11 + o]))

                g0, g1, g2 = ld01(0), ld01(1), ld01(2)
                for c in range(_C):
                    gn = ld01(c + 3) if c + 3 < _C else None
                    val = _lerp2(*g0, wu, wv)
                    val = jnp.where(ok, val, fail_c[c])
                    plsc.store_scatter(out_v,
                                       [pv, rowm[s] + (li * _C + c)], val)
                    g0, g1, g2 = g1, g2, gn
        return tuple(w for s in range(_NSUB)
                     for w in (subs[s][2][2][4], subs[s][2][2][5],
                               subs[s][2][3][4], subs[s][2][3][5],
                               jnp.where(subs[s][1], 1.0, 0.0)))

    def combine(w, pv):
        for s in range(_NSUB):
            wu2, wv2, wu3, wv3, okf = w[5 * s:5 * s + 5]
            ok = okf > 0.5
            for li, taps_r, wu, wv in ((2, taps2_v, wu2, wv2),
                                       (3, taps3_v, wu3, wv3)):
                def ldc(c, taps_r=taps_r):
                    return (plsc.load_gather(taps_r, [pv, trows[s][0], ccs[c]]),
                            plsc.load_gather(taps_r, [pv, trows[s][1], ccs[c]]),
                            plsc.load_gather(taps_r, [pv, trows[s][2], ccs[c]]),
                            plsc.load_gather(taps_r, [pv, trows[s][3], ccs[c]]))

                g0, g1, g2 = ldc(0), ldc(1), ldc(2)
                for c in range(_C):
                    gn = ldc(c + 3) if c + 3 < _C else None
                    val = _lerp2(*g0, wu, wv)
                    val = jnp.where(ok, val, fail_c[c])
                    plsc.store_scatter(out_v,
                                       [pv, rowm[s] + (li * _C + c)], val)
                    g0, g1, g2 = g1, g2, gn

    fire_in(0, 0)
    wait_in()
    subs0 = phase_a(zero16)
    fire_gathers(0, (sg2_a, sg3_a))
    w0 = l01(subs0, zero16)
    fire_in(1, 1)

    def loop(k, w):
        cur = k % 2
        nxt = 1 - cur
        pv_cur = zero16 + cur
        pv_nxt = zero16 + nxt

        @pl.when(k >= 1)
        def _():
            @pl.when(cur == 1)
            def _():
                wait_out(sout_a)
            @pl.when(cur == 0)
            def _():
                wait_out(sout_b)

        def prep(w_old):
            wait_in()
            subs = phase_a(pv_nxt)

            @pl.when(nxt == 0)
            def _():
                fire_gathers(0, (sg2_a, sg3_a))
            @pl.when(nxt == 1)
            def _():
                fire_gathers(1, (sg2_b, sg3_b))
            return l01(subs, pv_nxt)

        w_next = lax.cond(k < _NCHUNK - 1, prep, lambda w_old: w_old, w)

        @pl.when(cur == 0)
        def _():
            wait_gathers(0, (sg2_a, sg3_a))
            combine(w, zero16)
        @pl.when(cur == 1)
        def _():
            wait_gathers(1, (sg2_b, sg3_b))
            combine(w, zero16 + 1)

        @pl.when(cur == 0)
        def _():
            fire_out(k, 0, sout_a)
        @pl.when(cur == 1)
        def _():
            fire_out(k, 1, sout_b)

        @pl.when(k < _NCHUNK - 2)
        def _():
            fire_in(k + 2, cur)
        return w_next

    lax.fori_loop(0, _NCHUNK, loop, w0)
    wait_out(sout_b if (_NCHUNK - 1) % 2 == 1 else sout_a)


def kernel(inputs, params_0, params_1, params_2, params_3, fail_value):
    out, _, _ = _encode_sc(inputs.reshape(-1), params_0.reshape(-1),
                           params_1.reshape(-1), params_2, params_3,
                           fail_value)
    return out.reshape(_B, 4 * _C)


# speedup vs baseline: 1.0568x; 1.0005x over previous
import functools

import jax
import jax.numpy as jnp
from jax import lax
from jax.experimental import pallas as pl
from jax.experimental.pallas import tpu as pltpu
from jax.experimental.pallas import tpu_sc as plsc

_B = 262144
_C = 6
_RES = (8, 32, 128, 512)
_NC = 2
_NS = 16
_NW = _NC * _NS
_LANES = 16
_CHUNK = 64
_NSUB = _CHUNK // _LANES
_RPW = _B // _NW
_NCHUNK = _RPW // _CHUNK
_CP = 8
_R2 = 6 * _RES[2] * _RES[2]
_R3 = 6 * _RES[3] * _RES[3]
_T3CH = 2048
_T2CH = 1024
_N3 = (_RES[3] * _RES[3]) // (_NS * _T3CH)
_NCH3 = 6 * _N3


def _dir_math(x, y, z):
    ax, ay, az = jnp.abs(x), jnp.abs(y), jnp.abs(z)
    ma = jnp.maximum(jnp.maximum(ax, ay), az)
    is_x = (ax >= ay) & (ax >= az)
    is_y = (~is_x) & (ay >= az)
    face = jnp.where(
        is_x, jnp.where(x >= 0, 0, 1),
        jnp.where(is_y, jnp.where(y >= 0, 2, 3), jnp.where(z >= 0, 4, 5)),
    ).astype(jnp.int32)
    sc = jnp.where(is_x, jnp.where(x >= 0, -z, z),
                   jnp.where(is_y, x, jnp.where(z >= 0, x, -x)))
    tc = jnp.where(is_y, jnp.where(y >= 0, z, -z), -y)
    safe = jnp.where(ma > 0, ma, jnp.float32(1.0))
    u = 0.5 * (sc / safe + 1.0)
    v = 0.5 * (tc / safe + 1.0)
    return face, u, v, ma > 0


def _level_coords(u, v, L):
    Lf = jnp.float32(L)
    fu = jnp.clip(u * Lf - 0.5, 0.0, Lf - 1.0)
    fv = jnp.clip(v * Lf - 0.5, 0.0, Lf - 1.0)
    u0 = fu.astype(jnp.int32)
    v0 = fv.astype(jnp.int32)
    u1 = jnp.minimum(u0 + 1, L - 1)
    v1 = jnp.minimum(v0 + 1, L - 1)
    wu = fu - u0.astype(jnp.float32)
    wv = fv - v0.astype(jnp.float32)
    return u0, u1, v0, v1, wu, wv


def _lerp2(g00, g01, g10, g11, wu, wv):
    a = g00 + wu * (g01 - g00)
    b = g10 + wu * (g11 - g10)
    return a + wv * (b - a)


def _interleave(src_v, dst_v, iot, ccs):
    vrows, L = src_v.shape[1], src_v.shape[2]

    nj = L // _LANES

    def ldj(r, j):
        return [src_v[c, r, pl.ds(j * _LANES, _LANES)] for c in range(_C)]

    def irow(r, carry):
        rbase = iot + r * L
        g0, g1 = ldj(r, 0), ldj(r, 1)
        for j in range(nj):
            gn = ldj(r, j + 2) if j + 2 < nj else None
            rows = rbase + (j * _LANES)
            for c in range(_C):
                plsc.store_scatter(dst_v, [rows, ccs[c]], g0[c])
            g0, g1 = g1, gn
        return carry

    lax.fori_loop(0, vrows, irow, 0)


@functools.partial(
    pl.kernel,
    out_type=(jax.ShapeDtypeStruct((_B * 4 * _C,), jnp.float32),
              jax.ShapeDtypeStruct((_R2, _CP), jnp.float32),
              jax.ShapeDtypeStruct((_R3, _CP), jnp.float32)),
    mesh=plsc.VectorSubcoreMesh(core_axis_name="c", subcore_axis_name="s",
                                num_cores=_NC),
    compiler_params=pltpu.CompilerParams(needs_layout_passes=False,
                                         use_tc_tiling_on_sc=False),
    scratch_types=[
        pltpu.VMEM((6 * _C * _RES[0] * _RES[0],), jnp.float32),
        pltpu.VMEM((6 * _C * _RES[1] * _RES[1],), jnp.float32),
        pltpu.VMEM((2, _C, _T3CH // _RES[3], _RES[3]), jnp.float32),
        pltpu.VMEM((_C, _T2CH // _RES[2], _RES[2]), jnp.float32),
        pltpu.VMEM((2, _T3CH, _CP), jnp.float32),
        pltpu.VMEM((2, 3 * _CHUNK), jnp.float32),
        pltpu.VMEM((2, 4 * _CHUNK), jnp.int32),
        pltpu.VMEM((2, 4 * _CHUNK), jnp.int32),
        pltpu.VMEM((2, 4 * _CHUNK, _CP), jnp.float32),
        pltpu.VMEM((2, 4 * _CHUNK, _CP), jnp.float32),
        pltpu.VMEM((_C,), jnp.float32),
        pltpu.VMEM((2, _CHUNK * 4 * _C), jnp.float32),
        pltpu.SemaphoreType.DMA,
        pltpu.SemaphoreType.DMA,
        pltpu.SemaphoreType.DMA,
        pltpu.SemaphoreType.DMA,
        pltpu.SemaphoreType.DMA,
        pltpu.SemaphoreType.DMA,
        pltpu.SemaphoreType.DMA,
        pltpu.SemaphoreType.DMA,
        pltpu.SemaphoreType.DMA,
        pltpu.SemaphoreType.DMA,
        pltpu.SemaphoreType.DMA,
        pltpu.SemaphoreType.DMA,
    ],
)
def _encode_sc(inp_h, t0_h, t1_h, t2_h, t3_h, fail_h, out_h, t2r_h, t3r_h,
               t0_v, t1_v, pl_v, pl2_v, row_v, inp_v,
               idx2_v, idx3_v, taps2_v, taps3_v, fail_v, out_v,
               p1i_a, p1i_b, p1o_a, p1o_b, sin_a, sin_b,
               sg2_a, sg2_b, sg3_a, sg3_b, sout_a, sout_b):
    sid = lax.axis_index("s")
    wid = sid * _NC + lax.axis_index("c")
    iot = lax.iota(jnp.int32, _LANES)
    ccs = [jnp.full((_LANES,), c, jnp.int32) for c in range(_C)]
    p1i = (p1i_a, p1i_b)
    p1o = (p1o_a, p1o_b)
    sin = (sin_a, sin_b)
    sg = {2: (sg2_a, sg2_b), 3: (sg3_a, sg3_b)}
    sout = (sout_a, sout_b)
    tap_refs = {2: taps2_v, 3: taps3_v}
    idx_refs = {2: idx2_v, 3: idx3_v}
    src_refs = {2: t2r_h, 3: t3r_h}

    L3 = _RES[3]
    vrows3 = _T3CH // L3

    def p1_src(ci):
        f = ci // _N3
        k = ci % _N3
        v0 = sid * (vrows3 * _N3) + k * vrows3
        return t3_h.at[f, :, pl.ds(v0, vrows3), :], f * (L3 * L3) + v0 * L3

    def p1_fire_in(ci, par):
        src, _ = p1_src(ci)
        pltpu.async_copy(src, pl_v.at[par], p1i[par])

    def p1_step(ci, par, first):
        src, rb = p1_src(ci)
        pltpu.make_async_copy(src, pl_v.at[par], p1i[par]).wait()
        if not first:
            pltpu.make_async_copy(row_v.at[par],
                                  t3r_h.at[pl.ds(0, _T3CH)], p1o[par]).wait()
        _interleave(pl_v.at[par], row_v.at[par], iot, ccs)
        pltpu.async_copy(row_v.at[par], t3r_h.at[pl.ds(rb, _T3CH)], p1o[par])

    p1_fire_in(0, 0)
    p1_fire_in(1, 1)
    p1_step(0, 0, True)
    p1_fire_in(2, 0)
    p1_step(1, 1, True)
    p1_fire_in(3, 1)

    def p1_loop(kk, carry):
        ci = 2 + 2 * kk
        p1_step(ci, 0, False)
        p1_fire_in(ci + 2, 0)
        p1_step(ci + 1, 1, False)
        p1_fire_in(ci + 3, 1)
        return carry

    lax.fori_loop(0, (_NCH3 - 4) // 2, p1_loop, 0)
    p1_step(_NCH3 - 2, 0, False)
    p1_step(_NCH3 - 1, 1, False)
    pltpu.make_async_copy(row_v.at[0], t3r_h.at[pl.ds(0, _T3CH)], p1o[0]).wait()
    pltpu.make_async_copy(row_v.at[1], t3r_h.at[pl.ds(0, _T3CH)], p1o[1]).wait()

    L2 = _RES[2]
    vrows2 = _T2CH // L2

    def build2(f, carry):
        v0 = sid * vrows2
        pltpu.sync_copy(t2_h.at[f, :, pl.ds(v0, vrows2), :], pl2_v)
        _interleave(pl2_v, row_v.at[0], iot, ccs)
        rb = f * (L2 * L2) + v0 * L2
        pltpu.sync_copy(row_v.at[0, pl.ds(0, _T2CH)], t2r_h.at[pl.ds(rb, _T2CH)])
        return carry

    lax.fori_loop(0, 6, build2, 0)

    pltpu.sync_copy(t0_h, t0_v)
    pltpu.sync_copy(t1_h, t1_v)
    pltpu.sync_copy(fail_h, fail_v)
    plsc.subcore_barrier()

    base0 = wid * _RPW
    fail_c = [plsc.load_gather(fail_v, [ccs[c]]) for c in range(_C)]
    rowm = [(iot + s * _LANES) * (4 * _C) for s in range(_NSUB)]
    trows = [[iot + (t * _CHUNK + s * _LANES) for t in range(4)]
             for s in range(_NSUB)]
    zero16 = jnp.zeros((_LANES,), jnp.int32)

    def fire_in(ci, par):
        pltpu.async_copy(inp_h.at[pl.ds((base0 + ci * _CHUNK) * 3, 3 * _CHUNK)],
                         inp_v.at[par], sin_a)

    def wait_in():
        pltpu.make_async_copy(inp_h.at[pl.ds(0, 3 * _CHUNK)],
                              inp_v.at[0], sin_a).wait()

    def fire_gathers(par, sems):
        pltpu.async_copy(t2r_h.at[idx2_v.at[par]], taps2_v.at[par], sems[0])
        pltpu.async_copy(t3r_h.at[idx3_v.at[par]], taps3_v.at[par], sems[1])

    def wait_gathers(par, sems):
        pltpu.make_async_copy(t2r_h.at[idx2_v.at[par]],
                              taps2_v.at[par], sems[0]).wait()
        pltpu.make_async_copy(t3r_h.at[idx3_v.at[par]],
                              taps3_v.at[par], sems[1]).wait()

    def fire_out(ci, par, sem):
        pltpu.async_copy(out_v.at[par],
                         out_h.at[pl.ds((base0 + ci * _CHUNK) * 4 * _C,
                                        _CHUNK * 4 * _C)], sem)

    def wait_out(sem):
        pltpu.make_async_copy(out_v.at[0],
                              out_h.at[pl.ds(0, _CHUNK * 4 * _C)], sem).wait()

    def phase_a(pv):
        subs = []
        for s in range(_NSUB):
            c0s = iot * 3 + (s * 3 * _LANES)
            x = plsc.load_gather(inp_v, [pv, c0s])
            y = plsc.load_gather(inp_v, [pv, c0s + 1])
            z = plsc.load_gather(inp_v, [pv, c0s + 2])
            face, u, v, ok = _dir_math(x, y, z)
            lv = [_level_coords(u, v, L) for L in _RES]
            for li, idx_r in ((2, idx2_v), (3, idx3_v)):
                L = _RES[li]
                u0, u1, v0, v1, wu, wv = lv[li]
                fb = face * (L * L)
                r0 = fb + v0 * L
                r1 = fb + v1 * L
                taps = (r0 + u0, r0 + u1, r1 + u0, r1 + u1)
                for t in range(4):
                    plsc.store_scatter(
                        idx_r, [pv, iot + (t * _CHUNK + s * _LANES)], taps[t])
            subs.append((face, ok, lv))
        return subs

    def l01(subs, pv):
        for s in range(_NSUB):
            face, ok, lv = subs[s]
            for li, tv in ((0, t0_v), (1, t1_v)):
                L = _RES[li]
                u0, u1, v0, v1, wu, wv = lv[li]
                fb = face * (_C * L * L)
                a00 = fb + v0 * L + u0
                a01 = fb + v0 * L + u1
                a10 = fb + v1 * L + u0
                a11 = fb + v1 * L + u1
                def ld01(c):
                    o = c * (L * L)
                    return (plsc.load_gather(tv, [a00 + o]),
                            plsc.load_gather(tv, [a01 + o]),
                            plsc.load_gather(tv, [a10 + o]),
                            plsc.load_gather(tv, [a11 + o]))

                g0, g1, g2 = ld01(0), ld01(1), ld01(2)
                for c in range(_C):
                    gn = ld01(c + 3) if c + 3 < _C else None
                    val = _lerp2(*g0, wu, wv)
                    val = jnp.where(ok, val, fail_c[c])
                    plsc.store_scatter(out_v,
                                       [pv, rowm[s] + (li * _C + c)], val)
                    g0, g1, g2 = g1, g2, gn
        return tuple(w for s in range(_NSUB)
                     for w in (subs[s][2][2][4], subs[s][2][2][5],
                               subs[s][2][3][4], subs[s][2][3][5],
                               jnp.where(subs[s][1], 1.0, 0.0)))

    def combine(w, pv):
        for s in range(_NSUB):
            wu2, wv2, wu3, wv3, okf = w[5 * s:5 * s + 5]
            ok = okf > 0.5
            for li, taps_r, wu, wv in ((2, taps2_v, wu2, wv2),
                                       (3, taps3_v, wu3, wv3)):
                def ldc(c, taps_r=taps_r):
                    return (plsc.load_gather(taps_r, [pv, trows[s][0], ccs[c]]),
                            plsc.load_gather(taps_r, [pv, trows[s][1], ccs[c]]),
                            plsc.load_gather(taps_r, [pv, trows[s][2], ccs[c]]),
                            plsc.load_gather(taps_r, [pv, trows[s][3], ccs[c]]))

                g0, g1, g2 = ldc(0), ldc(1), ldc(2)
                for c in range(_C):
                    gn = ldc(c + 3) if c + 3 < _C else None
                    val = _lerp2(*g0, wu, wv)
                    val = jnp.where(ok, val, fail_c[c])
                    plsc.store_scatter(out_v,
                                       [pv, rowm[s] + (li * _C + c)], val)
                    g0, g1, g2 = g1, g2, gn

    fire_in(0, 0)
    wait_in()
    subs0 = phase_a(zero16)
    fire_gathers(0, (sg2_a, sg3_a))
    w0 = l01(subs0, zero16)
    fire_in(1, 1)

    def loop(k, w):
        cur = k % 2
        nxt = 1 - cur
        pv_cur = zero16 + cur
        pv_nxt = zero16 + nxt

        @pl.when(k >= 1)
        def _():
            @pl.when(cur == 1)
            def _():
                wait_out(sout_a)
            @pl.when(cur == 0)
            def _():
                wait_out(sout_b)

        def prep(w_old):
            wait_in()
            subs = phase_a(pv_nxt)

            @pl.when(nxt == 0)
            def _():
                fire_gathers(0, (sg2_a, sg3_a))
            @pl.when(nxt == 1)
            def _():
                fire_gathers(1, (sg2_b, sg3_b))
            return l01(subs, pv_nxt)

        w_next = lax.cond(k < _NCHUNK - 1, prep, lambda w_old: w_old, w)

        @pl.when(cur == 0)
        def _():
            wait_gathers(0, (sg2_a, sg3_a))
            combine(w, zero16)
        @pl.when(cur == 1)
        def _():
            wait_gathers(1, (sg2_b, sg3_b))
            combine(w, zero16 + 1)

        @pl.when(cur == 0)
        def _():
            fire_out(k, 0, sout_a)
        @pl.when(cur == 1)
        def _():
            fire_out(k, 1, sout_b)

        @pl.when(k < _NCHUNK - 2)
        def _():
            fire_in(k + 2, cur)
        return w_next

    lax.fori_loop(0, _NCHUNK, loop, w0)
    wait_out(sout_b if (_NCHUNK - 1) % 2 == 1 else sout_a)


def kernel(inputs, params_0, params_1, params_2, params_3, fail_value):
    out, _, _ = _encode_sc(inputs.reshape(-1), params_0.reshape(-1),
                           params_1.reshape(-1), params_2, params_3,
                           fail_value)
    return out.reshape(_B, 4 * _C)
